# baseline (speedup 1.0000x reference)
import jax
import jax.numpy as jnp
from jax import lax
from jax.experimental import pallas as pl
from jax.experimental.pallas import tpu as pltpu
from jax.experimental.pallas import tpu_sc as plsc

_N = 100000
_E = 6400000
_R = 16
_H = 2
_NH = 4

_NP = 102400
_TS = _NP // 16
_CW = 5120
_NCH = _E // _CW
_KMAX = -(-_NCH // 32)
_NSEG = 4
_SEG = _TS // _NSEG


def _make_fused_kernel():
    out_type = (jax.ShapeDtypeStruct((2 * _NP,), jnp.float32),
                jax.ShapeDtypeStruct((2 * _NP,), jnp.float32),
                jax.ShapeDtypeStruct((2 * _NP,), jnp.float32))

    scratch = [
        pltpu.VMEM_SHARED((_NP,), jnp.int32),
        pltpu.VMEM_SHARED((_NP,), jnp.float32),
        pltpu.VMEM_SHARED((_NP,), jnp.float32),
        pltpu.VMEM((_CW,), jnp.int32),
        pltpu.VMEM((_CW,), jnp.int32),
        pltpu.VMEM((_CW,), jnp.int32),
        pltpu.VMEM((_CW,), jnp.int32),
        pltpu.VMEM((2, _CW), jnp.int32),
        pltpu.VMEM((2, _CW), jnp.int32),
        pltpu.VMEM((_CW,), jnp.int32),
        pltpu.VMEM((_CW,), jnp.int32),
        pltpu.VMEM((_CW,), jnp.float32),
        pltpu.VMEM((_CW,), jnp.float32),
        pltpu.VMEM((_CW,), jnp.float32),
        pltpu.VMEM((_CW,), jnp.float32),
        pltpu.VMEM((16, 16), jnp.float32),
        pltpu.VMEM((8, 16), jnp.float32),
        pltpu.VMEM((_SEG,), jnp.float32),
        pltpu.VMEM((_SEG,), jnp.float32),
        pltpu.VMEM((_SEG,), jnp.int32),
        pltpu.VMEM((_SEG,), jnp.float32),
        pltpu.VMEM((_SEG,), jnp.float32),
        pltpu.VMEM((_SEG,), jnp.float32),
        pltpu.VMEM((_SEG,), jnp.float32),
        pltpu.VMEM((_SEG,), jnp.float32),
        pltpu.VMEM((2 * _SEG,), jnp.float32),
        pltpu.SemaphoreType.REGULAR,
        pltpu.SemaphoreType.DMA,
        pltpu.SemaphoreType.DMA,
        pltpu.SemaphoreType.DMA,
        pltpu.SemaphoreType.DMA,
    ]

    mesh = plsc.VectorSubcoreMesh(core_axis_name="c", subcore_axis_name="s")

    def body(src_h, dst_h, tn_h, wtab_h, bias_h, xf_h, hxA, hxB, xout,
             xq, h0s, h1s, srcb0, srcb1, dstb0, dstb1, tnb0, tnb1,
             xsqa, xsqb, m0a, m1a, m0b, m1b, wbuf, bbuf,
             xb0, xb1, xqb, zb, rh0, rh1, oh0, oh1, fbuf,
             xsem, lsem, dsem, gsem, ssem):
        c = lax.axis_index("c")
        s = lax.axis_index("s")
        wid = s * 2 + c
        nbase = s * _TS
        iot = lax.iota(jnp.int32, 16)
        zero = jnp.zeros((16,), jnp.float32)

        pltpu.sync_copy(wtab_h, wbuf)
        pltpu.sync_copy(bias_h, bbuf)

        srcbs = (srcb0, srcb1)
        dstbs = (dstb0, dstb1)
        tnbs = (tnb0, tnb1)
        xsqs = (xsqa, xsqb)
        m0s = (m0a, m0b)
        m1s = (m1a, m1b)

        def issue_stn(cidx, b):
            pltpu.async_copy(src_h.at[cidx], srcbs[b], lsem)
            pltpu.async_copy(tn_h.at[cidx], tnbs[b], lsem)

        def wait_stn(cidx, b):
            pltpu.make_async_copy(src_h.at[cidx], srcbs[b], lsem).wait()
            pltpu.make_async_copy(tn_h.at[cidx], tnbs[b], lsem).wait()

        def issue_dst(cidx, b):
            pltpu.async_copy(dst_h.at[cidx], dstbs[b], dsem)

        def wait_dst(cidx, b):
            pltpu.make_async_copy(dst_h.at[cidx], dstbs[b], dsem).wait()

        def issue_gathers(b):
            pltpu.async_copy(xq.at[srcbs[b]], xsqs[b], gsem)

        def wait_gathers(b):
            pltpu.make_async_copy(xq.at[srcbs[b]], xsqs[b], gsem).wait()

        def issue_scatters(b):
            pltpu.async_copy(m0s[b], h0s.at[dstbs[b]], ssem, add=True)
            pltpu.async_copy(m1s[b], h1s.at[dstbs[b]], ssem, add=True)

        def wait_scatters(b):
            pltpu.make_async_copy(m0s[b], h0s.at[dstbs[b]], ssem).wait()
            pltpu.make_async_copy(m1s[b], h1s.at[dstbs[b]], ssem).wait()

        def prologue(layer):
            if layer > 0:
                b0v = bbuf[2 * (layer - 1), :]
                b1v = bbuf[2 * (layer - 1) + 1, :]
            for seg in range(_NSEG):
                sbase = nbase + seg * _SEG
                if layer in (0, 2):
                    pltpu.sync_copy(xf_h.at[pl.ds(sbase * 2, 2 * _SEG)],
                                    fbuf)
                if layer > 0:
                    pltpu.sync_copy(h0s.at[pl.ds(sbase, _SEG)], oh0)
                    pltpu.sync_copy(h1s.at[pl.ds(sbase, _SEG)], oh1)

                    @pl.when(c == 0)
                    def _():
                        pltpu.sync_copy(hxB.at[pl.ds(sbase, _SEG)], rh0)
                        pltpu.sync_copy(hxB.at[pl.ds(_NP + sbase, _SEG)],
                                        rh1)

                    @pl.when(c == 1)
                    def _():
                        pltpu.sync_copy(hxA.at[pl.ds(sbase, _SEG)], rh0)
                        pltpu.sync_copy(hxA.at[pl.ds(_NP + sbase, _SEG)],
                                        rh1)

                def pro_body(j, _):
                    off = j * 16
                    if layer in (0, 2):
                        f0 = plsc.load_gather(fbuf, [(off + iot) * 2])
                        f1 = plsc.load_gather(fbuf, [(off + iot) * 2 + 1])
                    if layer == 0:
                        x0v, x1v = f0, f1
                    else:
                        x0v = oh0[pl.ds(off, 16)] + rh0[pl.ds(off, 16)] + b0v
                        x1v = oh1[pl.ds(off, 16)] + rh1[pl.ds(off, 16)] + b1v
                        x0v = jnp.maximum(x0v, 0.0)
                        x1v = jnp.maximum(x1v, 0.0)
                        if layer == 2:
                            x0v = x0v + f0
                            x1v = x1v + f1
                    b0i = plsc.bitcast(x0v, jnp.int32)
                    b1i = plsc.bitcast(x1v, jnp.int32)
                    q0 = b0i + 32767 + ((b0i >> 16) & 1)
                    q1 = b1i + 32767 + ((b1i >> 16) & 1)
                    xqb[pl.ds(off, 16)] = (
                        lax.shift_right_logical(q0, 16)
                        | (q1 & jnp.int32(-65536)))
                    if layer == 2:
                        xb0[pl.ds(off, 16)] = x0v
                        xb1[pl.ds(off, 16)] = x1v
                    if layer == 0 and seg == 0:
                        zb[pl.ds(off, 16)] = zero
                    return 0

                lax.fori_loop(0, _SEG // 16, pro_body, 0)
                pltpu.sync_copy(xqb, xq.at[pl.ds(sbase, _SEG)])
                pltpu.sync_copy(zb, h0s.at[pl.ds(sbase, _SEG)])
                pltpu.sync_copy(zb, h1s.at[pl.ds(sbase, _SEG)])
                if layer == 2:
                    @pl.when(c == 0)
                    def _():
                        pltpu.sync_copy(xb0, xout.at[pl.ds(sbase, _SEG)])
                        pltpu.sync_copy(xb1,
                                        xout.at[pl.ds(_NP + sbase, _SEG)])
            plsc.subcore_barrier()

        def edge_phase(layer):
            wrow = 4 * layer

            issue_stn(wid, 0)
            issue_dst(wid, 0)
            wait_stn(wid, 0)
            issue_gathers(0)

            @pl.when(wid + 32 < _NCH)
            def _():
                issue_stn(wid + 32, 1)

            def chunk_pair(g, _):
                for b in (0, 1):
                    k = g * 2 + b
                    cidx = wid + 32 * k
                    tnb = tnbs[b]
                    xsq = xsqs[b]
                    m0, m1 = m0s[b], m1s[b]

                    @pl.when(cidx < _NCH)
                    def _():
                        wait_gathers(b)
                        ncidx = cidx + 32

                        @pl.when(ncidx < _NCH)
                        def _():
                            wait_stn(ncidx, 1 - b)
                            issue_gathers(1 - b)

                        def ed_body(jj, _):
                            for sub in range(4):
                                off = jj * 64 + sub * 16
                                t = tnb[0, pl.ds(off, 16)]
                                nv = plsc.bitcast(tnb[1, pl.ds(off, 16)],
                                                  jnp.float32)
                                qv = xsq[pl.ds(off, 16)]
                                a0 = plsc.bitcast(
                                    lax.shift_left(qv, 16), jnp.float32)
                                a1 = plsc.bitcast(
                                    qv & jnp.int32(-65536), jnp.float32)
                                w00 = plsc.load_gather(
                                    wbuf.at[wrow + 0], [t])
                                w10 = plsc.load_gather(
                                    wbuf.at[wrow + 1], [t])
                                w01 = plsc.load_gather(
                                    wbuf.at[wrow + 2], [t])
                                w11 = plsc.load_gather(
                                    wbuf.at[wrow + 3], [t])
                                na0 = a0 * nv
                                na1 = a1 * nv
                                m0[pl.ds(off, 16)] = na0 * w00 + na1 * w10
                                m1[pl.ds(off, 16)] = na0 * w01 + na1 * w11
                            return 0

                        lax.fori_loop(0, _CW // 64, ed_body, 0)

                        @pl.when(cidx + 64 < _NCH)
                        def _():
                            issue_stn(cidx + 64, b)

                        @pl.when(cidx >= 32)
                        def _():
                            wait_scatters(1 - b)

                        wait_dst(cidx, b)
                        issue_scatters(b)

                        @pl.when(ncidx < _NCH)
                        def _():
                            issue_dst(ncidx, 1 - b)
                return 0

            lax.fori_loop(0, _KMAX // 2, chunk_pair, 0)

            last1 = wid + 32 * (_KMAX - 1)
            last2 = wid + 32 * (_KMAX - 2)

            @pl.when(last1 < _NCH)
            def _():
                wait_scatters((_KMAX - 1) % 2)

            @pl.when((last2 < _NCH) & (last1 >= _NCH))
            def _():
                wait_scatters((_KMAX - 2) % 2)

            plsc.subcore_barrier()

        def write_partials():
            @pl.when(c == 0)
            def _():
                pltpu.sync_copy(h0s.at[pl.ds(nbase, _TS)],
                                hxA.at[pl.ds(nbase, _TS)])
                pltpu.sync_copy(h1s.at[pl.ds(nbase, _TS)],
                                hxA.at[pl.ds(_NP + nbase, _TS)])

            @pl.when(c == 1)
            def _():
                pltpu.sync_copy(h0s.at[pl.ds(nbase, _TS)],
                                hxB.at[pl.ds(nbase, _TS)])
                pltpu.sync_copy(h1s.at[pl.ds(nbase, _TS)],
                                hxB.at[pl.ds(_NP + nbase, _TS)])
            plsc.subcore_barrier()

        def rendezvous():
            pl.semaphore_signal(xsem, 1, core_index=1 - c)
            pl.semaphore_wait(xsem, 1)
            plsc.subcore_barrier()

        for layer in range(_NH):
            prologue(layer)
            if layer > 0:
                rendezvous()
            edge_phase(layer)
            write_partials()
            if layer < _NH - 1:
                rendezvous()

    return pl.kernel(body, out_type=out_type,
                     compiler_params=pltpu.CompilerParams(
                         needs_layout_passes=False),
                     mesh=mesh, scratch_types=scratch)


def _mlp_body(hp_ref, x2_ref, wc_ref, b3_ref, mb_ref, out_ref):
    x40 = hp_ref[0, :] + hp_ref[2, :] + b3_ref[0, 0] + x2_ref[0, :]
    x41 = hp_ref[1, :] + hp_ref[3, :] + b3_ref[0, 1] + x2_ref[1, :]
    acc = jnp.sum(x40 * wc_ref[0, :]) + jnp.sum(x41 * wc_ref[1, :])
    z = acc + mb_ref[0, 0]
    out_ref[0, 0] = 1.0 / (1.0 + jnp.exp(-z))


def kernel(features, edge_index, edge_type, norm, bases, w_comp, layer_bias,
           mlp_w, mlp_b):
    src2 = edge_index[0].reshape(_NCH, _CW)
    dst2 = edge_index[1].reshape(_NCH, _CW)
    tn = jnp.stack(
        [edge_type.reshape(_NCH, _CW),
         jax.lax.bitcast_convert_type(norm.reshape(_NCH, _CW), jnp.int32)],
        axis=1)
    xf = jnp.pad(features.reshape(-1), (0, 2 * _NP - 2 * _N))

    W = jnp.einsum("lrb,lbio->lrio", w_comp, bases)
    wtabs = jnp.stack(
        [W[:, :, 0, 0], W[:, :, 1, 0], W[:, :, 0, 1], W[:, :, 1, 1]], axis=1
    ).reshape(_NH * 4, _R)
    biases = jnp.broadcast_to(
        layer_bias[:, :, None], (_NH, _H, 16)).reshape(_NH * 2, 16)

    kf = _make_fused_kernel()
    hxa, hxb, x2 = kf(src2, dst2, tn, wtabs, biases, xf)

    h3 = jnp.concatenate([hxa, hxb]).reshape(4, _NP)
    wcols = jnp.pad(mlp_w.reshape(_N, _H).T, ((0, 0), (0, _NP - _N)))
    b3 = layer_bias[3].reshape(1, _H)
    mb = mlp_b.reshape(1, 1)
    out = pl.pallas_call(
        _mlp_body,
        out_shape=jax.ShapeDtypeStruct((1, 1), jnp.float32),
        in_specs=[pl.BlockSpec(memory_space=pltpu.VMEM)] * 3
        + [pl.BlockSpec(memory_space=pltpu.SMEM)] * 2,
        out_specs=pl.BlockSpec(memory_space=pltpu.SMEM),
    )(h3, x2.reshape(2, _NP), wcols, b3, mb)
    return out.reshape(1, 1)


# speedup vs baseline: 1.1349x; 1.1349x over previous
import jax
import jax.numpy as jnp
from jax import lax
from jax.experimental import pallas as pl
from jax.experimental.pallas import tpu as pltpu
from jax.experimental.pallas import tpu_sc as plsc

_N = 100000
_E = 6400000
_R = 16
_H = 2
_NH = 4

_NP = 102400
_TS = _NP // 16
_CW = 5120
_NCH = _E // _CW
_KMAX = -(-_NCH // 32)
_NSEG = 4
_SEG = _TS // _NSEG


def _make_fused_kernel():
    out_type = (jax.ShapeDtypeStruct((2 * _NP,), jnp.float32),
                jax.ShapeDtypeStruct((2 * _NP,), jnp.float32),
                jax.ShapeDtypeStruct((2 * _NP,), jnp.float32))

    scratch = [
        pltpu.VMEM_SHARED((_NP,), jnp.float32),
        pltpu.VMEM_SHARED((_NP,), jnp.float32),
        pltpu.VMEM_SHARED((_NP,), jnp.float32),
        pltpu.VMEM_SHARED((_NP,), jnp.float32),
        pltpu.VMEM((_CW,), jnp.int32),
        pltpu.VMEM((_CW,), jnp.int32),
        pltpu.VMEM((_CW,), jnp.int32),
        pltpu.VMEM((_CW,), jnp.int32),
        pltpu.VMEM((2, _CW), jnp.int32),
        pltpu.VMEM((2, _CW), jnp.int32),
        pltpu.VMEM((_CW,), jnp.float32),
        pltpu.VMEM((_CW,), jnp.float32),
        pltpu.VMEM((_CW,), jnp.float32),
        pltpu.VMEM((_CW,), jnp.float32),
        pltpu.VMEM((_CW,), jnp.float32),
        pltpu.VMEM((_CW,), jnp.float32),
        pltpu.VMEM((_CW,), jnp.float32),
        pltpu.VMEM((_CW,), jnp.float32),
        pltpu.VMEM((16, 16), jnp.float32),
        pltpu.VMEM((8, 16), jnp.float32),
        pltpu.VMEM((_SEG,), jnp.float32),
        pltpu.VMEM((_SEG,), jnp.float32),
        pltpu.VMEM((_SEG,), jnp.float32),
        pltpu.VMEM((_SEG,), jnp.float32),
        pltpu.VMEM((_SEG,), jnp.float32),
        pltpu.VMEM((_SEG,), jnp.float32),
        pltpu.VMEM((_SEG,), jnp.float32),
        pltpu.VMEM((2 * _SEG,), jnp.float32),
        pltpu.SemaphoreType.REGULAR,
        pltpu.SemaphoreType.DMA,
        pltpu.SemaphoreType.DMA,
        pltpu.SemaphoreType.DMA,
        pltpu.SemaphoreType.DMA,
    ]

    mesh = plsc.VectorSubcoreMesh(core_axis_name="c", subcore_axis_name="s")

    def body(src_h, dst_h, tn_h, wtab_h, bias_h, xf_h, hxA, hxB, xout,
             x0s, x1s, h0s, h1s, srcb0, srcb1, dstb0, dstb1, tnb0, tnb1,
             xs0a, xs1a, xs0b, xs1b, m0a, m1a, m0b, m1b, wbuf, bbuf,
             xb0, xb1, zb, rh0, rh1, oh0, oh1, fbuf,
             xsem, lsem, dsem, gsem, ssem):
        c = lax.axis_index("c")
        s = lax.axis_index("s")
        wid = s * 2 + c
        nbase = s * _TS
        iot = lax.iota(jnp.int32, 16)
        zero = jnp.zeros((16,), jnp.float32)

        pltpu.sync_copy(wtab_h, wbuf)
        pltpu.sync_copy(bias_h, bbuf)

        srcbs = (srcb0, srcb1)
        dstbs = (dstb0, dstb1)
        tnbs = (tnb0, tnb1)
        xs0s = (xs0a, xs0b)
        xs1s = (xs1a, xs1b)
        m0s = (m0a, m0b)
        m1s = (m1a, m1b)

        def issue_stn(cidx, b):
            pltpu.async_copy(src_h.at[cidx], srcbs[b], lsem)
            pltpu.async_copy(tn_h.at[cidx], tnbs[b], lsem)

        def wait_stn(cidx, b):
            pltpu.make_async_copy(src_h.at[cidx], srcbs[b], lsem).wait()
            pltpu.make_async_copy(tn_h.at[cidx], tnbs[b], lsem).wait()

        def issue_dst(cidx, b):
            pltpu.async_copy(dst_h.at[cidx], dstbs[b], dsem)

        def wait_dst(cidx, b):
            pltpu.make_async_copy(dst_h.at[cidx], dstbs[b], dsem).wait()

        def issue_gathers(b):
            pltpu.async_copy(x0s.at[srcbs[b]], xs0s[b], gsem)
            pltpu.async_copy(x1s.at[srcbs[b]], xs1s[b], gsem)

        def wait_gathers(b):
            pltpu.make_async_copy(x0s.at[srcbs[b]], xs0s[b], gsem).wait()
            pltpu.make_async_copy(x1s.at[srcbs[b]], xs1s[b], gsem).wait()

        def issue_scatters(b):
            pltpu.async_copy(m0s[b], h0s.at[dstbs[b]], ssem, add=True)
            pltpu.async_copy(m1s[b], h1s.at[dstbs[b]], ssem, add=True)

        def wait_scatters(b):
            pltpu.make_async_copy(m0s[b], h0s.at[dstbs[b]], ssem).wait()
            pltpu.make_async_copy(m1s[b], h1s.at[dstbs[b]], ssem).wait()

        def prologue(layer):
            if layer > 0:
                b0v = bbuf[2 * (layer - 1), :]
                b1v = bbuf[2 * (layer - 1) + 1, :]
            for seg in range(_NSEG):
                sbase = nbase + seg * _SEG
                if layer in (0, 2):
                    pltpu.sync_copy(xf_h.at[pl.ds(sbase * 2, 2 * _SEG)],
                                    fbuf)
                if layer > 0:
                    pltpu.sync_copy(h0s.at[pl.ds(sbase, _SEG)], oh0)
                    pltpu.sync_copy(h1s.at[pl.ds(sbase, _SEG)], oh1)

                    @pl.when(c == 0)
                    def _():
                        pltpu.sync_copy(hxB.at[pl.ds(sbase, _SEG)], rh0)
                        pltpu.sync_copy(hxB.at[pl.ds(_NP + sbase, _SEG)],
                                        rh1)

                    @pl.when(c == 1)
                    def _():
                        pltpu.sync_copy(hxA.at[pl.ds(sbase, _SEG)], rh0)
                        pltpu.sync_copy(hxA.at[pl.ds(_NP + sbase, _SEG)],
                                        rh1)

                def pro_body(j, _):
                    off = j * 16
                    if layer in (0, 2):
                        f0 = plsc.load_gather(fbuf, [(off + iot) * 2])
                        f1 = plsc.load_gather(fbuf, [(off + iot) * 2 + 1])
                    if layer == 0:
                        x0v, x1v = f0, f1
                    else:
                        x0v = oh0[pl.ds(off, 16)] + rh0[pl.ds(off, 16)] + b0v
                        x1v = oh1[pl.ds(off, 16)] + rh1[pl.ds(off, 16)] + b1v
                        x0v = jnp.maximum(x0v, 0.0)
                        x1v = jnp.maximum(x1v, 0.0)
                        if layer == 2:
                            x0v = x0v + f0
                            x1v = x1v + f1
                    xb0[pl.ds(off, 16)] = x0v
                    xb1[pl.ds(off, 16)] = x1v
                    if layer == 0 and seg == 0:
                        zb[pl.ds(off, 16)] = zero
                    return 0

                lax.fori_loop(0, _SEG // 16, pro_body, 0)
                pltpu.sync_copy(xb0, x0s.at[pl.ds(sbase, _SEG)])
                pltpu.sync_copy(xb1, x1s.at[pl.ds(sbase, _SEG)])
                pltpu.sync_copy(zb, h0s.at[pl.ds(sbase, _SEG)])
                pltpu.sync_copy(zb, h1s.at[pl.ds(sbase, _SEG)])
                if layer == 2:
                    @pl.when(c == 0)
                    def _():
                        pltpu.sync_copy(xb0, xout.at[pl.ds(sbase, _SEG)])
                        pltpu.sync_copy(xb1,
                                        xout.at[pl.ds(_NP + sbase, _SEG)])
            plsc.subcore_barrier()

        def edge_phase(layer):
            wrow = 4 * layer

            issue_stn(wid, 0)
            issue_dst(wid, 0)
            wait_stn(wid, 0)
            issue_gathers(0)

            @pl.when(wid + 32 < _NCH)
            def _():
                issue_stn(wid + 32, 1)

            def chunk_pair(g, _):
                for b in (0, 1):
                    k = g * 2 + b
                    cidx = wid + 32 * k
                    tnb = tnbs[b]
                    xs0, xs1 = xs0s[b], xs1s[b]
                    m0, m1 = m0s[b], m1s[b]

                    @pl.when(cidx < _NCH)
                    def _():
                        wait_gathers(b)
                        ncidx = cidx + 32

                        @pl.when(ncidx < _NCH)
                        def _():
                            wait_stn(ncidx, 1 - b)
                            issue_gathers(1 - b)

                        def ed_body(jj, _):
                            for sub in range(8):
                                off = jj * 128 + sub * 16
                                t = tnb[0, pl.ds(off, 16)]
                                nv = plsc.bitcast(tnb[1, pl.ds(off, 16)],
                                                  jnp.float32)
                                a0 = xs0[pl.ds(off, 16)]
                                a1 = xs1[pl.ds(off, 16)]
                                w00 = plsc.load_gather(
                                    wbuf.at[wrow + 0], [t])
                                w10 = plsc.load_gather(
                                    wbuf.at[wrow + 1], [t])
                                w01 = plsc.load_gather(
                                    wbuf.at[wrow + 2], [t])
                                w11 = plsc.load_gather(
                                    wbuf.at[wrow + 3], [t])
                                na0 = a0 * nv
                                na1 = a1 * nv
                                m0[pl.ds(off, 16)] = na0 * w00 + na1 * w10
                                m1[pl.ds(off, 16)] = na0 * w01 + na1 * w11
                            return 0

                        lax.fori_loop(0, _CW // 128, ed_body, 0)

                        @pl.when(cidx + 64 < _NCH)
                        def _():
                            issue_stn(cidx + 64, b)

                        @pl.when(cidx >= 32)
                        def _():
                            wait_scatters(1 - b)

                        wait_dst(cidx, b)
                        issue_scatters(b)

                        @pl.when(ncidx < _NCH)
                        def _():
                            issue_dst(ncidx, 1 - b)
                return 0

            lax.fori_loop(0, _KMAX // 2, chunk_pair, 0)

            last1 = wid + 32 * (_KMAX - 1)
            last2 = wid + 32 * (_KMAX - 2)

            @pl.when(last1 < _NCH)
            def _():
                wait_scatters((_KMAX - 1) % 2)

            @pl.when((last2 < _NCH) & (last1 >= _NCH))
            def _():
                wait_scatters((_KMAX - 2) % 2)

            plsc.subcore_barrier()

        def write_partials():
            @pl.when(c == 0)
            def _():
                pltpu.sync_copy(h0s.at[pl.ds(nbase, _TS)],
                                hxA.at[pl.ds(nbase, _TS)])
                pltpu.sync_copy(h1s.at[pl.ds(nbase, _TS)],
                                hxA.at[pl.ds(_NP + nbase, _TS)])

            @pl.when(c == 1)
            def _():
                pltpu.sync_copy(h0s.at[pl.ds(nbase, _TS)],
                                hxB.at[pl.ds(nbase, _TS)])
                pltpu.sync_copy(h1s.at[pl.ds(nbase, _TS)],
                                hxB.at[pl.ds(_NP + nbase, _TS)])
            plsc.subcore_barrier()

        def rendezvous():
            pl.semaphore_signal(xsem, 1, core_index=1 - c)
            pl.semaphore_wait(xsem, 1)
            plsc.subcore_barrier()

        for layer in range(_NH):
            prologue(layer)
            if layer > 0:
                rendezvous()
            edge_phase(layer)
            write_partials()
            if layer < _NH - 1:
                rendezvous()

    return pl.kernel(body, out_type=out_type,
                     compiler_params=pltpu.CompilerParams(
                         needs_layout_passes=False),
                     mesh=mesh, scratch_types=scratch)


def _mlp_body(hp_ref, x2_ref, wc_ref, b3_ref, mb_ref, out_ref):
    x40 = hp_ref[0, :] + hp_ref[2, :] + b3_ref[0, 0] + x2_ref[0, :]
    x41 = hp_ref[1, :] + hp_ref[3, :] + b3_ref[0, 1] + x2_ref[1, :]
    acc = jnp.sum(x40 * wc_ref[0, :]) + jnp.sum(x41 * wc_ref[1, :])
    z = acc + mb_ref[0, 0]
    out_ref[0, 0] = 1.0 / (1.0 + jnp.exp(-z))


def kernel(features, edge_index, edge_type, norm, bases, w_comp, layer_bias,
           mlp_w, mlp_b):
    src2 = edge_index[0].reshape(_NCH, _CW)
    dst2 = edge_index[1].reshape(_NCH, _CW)
    tn = jnp.stack(
        [edge_type.reshape(_NCH, _CW),
         jax.lax.bitcast_convert_type(norm.reshape(_NCH, _CW), jnp.int32)],
        axis=1)
    xf = jnp.pad(features.reshape(-1), (0, 2 * _NP - 2 * _N))

    W = jnp.einsum("lrb,lbio->lrio", w_comp, bases)
    wtabs = jnp.stack(
        [W[:, :, 0, 0], W[:, :, 1, 0], W[:, :, 0, 1], W[:, :, 1, 1]], axis=1
    ).reshape(_NH * 4, _R)
    biases = jnp.broadcast_to(
        layer_bias[:, :, None], (_NH, _H, 16)).reshape(_NH * 2, 16)

    kf = _make_fused_kernel()
    hxa, hxb, x2 = kf(src2, dst2, tn, wtabs, biases, xf)

    h3 = jnp.concatenate([hxa, hxb]).reshape(4, _NP)
    wcols = jnp.pad(mlp_w.reshape(_N, _H).T, ((0, 0), (0, _NP - _N)))
    b3 = layer_bias[3].reshape(1, _H)
    mb = mlp_b.reshape(1, 1)
    out = pl.pallas_call(
        _mlp_body,
        out_shape=jax.ShapeDtypeStruct((1, 1), jnp.float32),
        in_specs=[pl.BlockSpec(memory_space=pltpu.VMEM)] * 3
        + [pl.BlockSpec(memory_space=pltpu.SMEM)] * 2,
        out_specs=pl.BlockSpec(memory_space=pltpu.SMEM),
    )(h3, x2.reshape(2, _NP), wcols, b3, mb)
    return out.reshape(1, 1)
